# baseline (speedup 1.0000x reference)
import jax
import jax.numpy as jnp
from jax.experimental import pallas as pl
from jax.experimental.pallas import tpu as pltpu

_BM = 400


def _gcn_kern(adj_ref, seq_ref, w_ref, bias_ref, alpha_ref, o_ref,
              f_ref, h1_ref):
    t = pl.program_id(0)
    nb = (pl.num_programs(0) + 1) // 2

    @pl.when(t == 0)
    def _():
        f_ref[...] = jax.lax.dot_general(
            seq_ref[...], w_ref[...],
            (((1,), (1,)), ((), ())),
            preferred_element_type=jnp.float32,
        )

    @pl.when(t < nb)
    def _():
        h1_ref[pl.ds(t * _BM, _BM), :] = jnp.dot(
            adj_ref[...], f_ref[...], preferred_element_type=jnp.float32)

    @pl.when(t >= nb - 1)
    def _():
        v = jnp.dot(adj_ref[...], h1_ref[...],
                    preferred_element_type=jnp.float32)
        v = v + bias_ref[...]
        o_ref[...] = jnp.where(v >= 0, v, alpha_ref[0, 0] * v)


def kernel(seq, adj, W_fc, bias, prelu_a):
    n, in_ft = seq.shape
    out_ft = W_fc.shape[0]
    nb = n // _BM

    def adj_idx(t):
        return (jnp.where(t < nb, t, 2 * nb - 2 - t), 0)

    def out_idx(t):
        return (jnp.where(t < nb, nb - 1, 2 * nb - 2 - t), 0)

    return pl.pallas_call(
        _gcn_kern,
        grid=(2 * nb - 1,),
        in_specs=[
            pl.BlockSpec((_BM, n), adj_idx),
            pl.BlockSpec((n, in_ft), lambda t: (0, 0)),
            pl.BlockSpec((out_ft, in_ft), lambda t: (0, 0)),
            pl.BlockSpec((1, out_ft), lambda t: (0, 0)),
            pl.BlockSpec((1, 1), lambda t: (0, 0)),
        ],
        out_specs=pl.BlockSpec((_BM, out_ft), out_idx),
        out_shape=jax.ShapeDtypeStruct((n, out_ft), jnp.float32),
        scratch_shapes=[
            pltpu.VMEM((n, out_ft), jnp.float32),
            pltpu.VMEM((n, out_ft), jnp.float32),
        ],
        compiler_params=pltpu.CompilerParams(
            dimension_semantics=("arbitrary",),
        ),
    )(adj, seq, W_fc, bias.reshape(1, out_ft), prelu_a.reshape(1, 1))
